# Initial kernel scaffold; baseline (speedup 1.0000x reference)
import jax
import jax.numpy as jnp
from jax.experimental import pallas as pl


def kernel(pred, label):
    raise NotImplementedError("write your pallas kernel here")


# speedup vs baseline: 1.6690x; 1.6690x over previous
import jax
import jax.numpy as jnp
from jax import lax
from jax.experimental import pallas as pl
from jax.experimental.pallas import tpu as pltpu

_FACTOR = 3
_IGNORE = -100
_N = 262144
_C = 21
_BLK = 2048
_G = _N // _BLK


def _ce_pass_body(pred_ref, label_ref, pos_ref, lse_ref, picked_ref):
    x = pred_ref[...]
    lab = label_ref[0, 0, :]
    m = jnp.max(x)
    e = jnp.exp(x - m)
    s = jnp.sum(e, axis=1)
    lse = jnp.log(s) + m
    iota = lax.broadcasted_iota(jnp.int32, x.shape, 1)
    picked = jnp.sum(jnp.where(iota == lab[:, None], x, 0.0), axis=1)
    i = pl.program_id(0)
    pos_ref[0, i] = jnp.sum((lab != 0).astype(jnp.int32))
    lse_ref[0, i] = jnp.sum(lse)
    picked_ref[0, i] = jnp.sum(picked)


def _ce_pass(pred, label):
    label3 = label.reshape(_G, 1, _BLK)
    out = pl.pallas_call(
        _ce_pass_body,
        grid=(_G,),
        in_specs=[
            pl.BlockSpec((_BLK, _C), lambda i: (i, 0)),
            pl.BlockSpec((1, 1, _BLK), lambda i: (i, 0, 0)),
        ],
        out_specs=[
            pl.BlockSpec((1, _G), lambda i: (0, 0), memory_space=pltpu.SMEM),
            pl.BlockSpec((1, _G), lambda i: (0, 0), memory_space=pltpu.SMEM),
            pl.BlockSpec((1, _G), lambda i: (0, 0), memory_space=pltpu.SMEM),
        ],
        out_shape=[
            jax.ShapeDtypeStruct((1, _G), jnp.int32),
            jax.ShapeDtypeStruct((1, _G), jnp.float32),
            jax.ShapeDtypeStruct((1, _G), jnp.float32),
        ],
    )(pred, label3)
    pos_parts, lse_parts, picked_parts = out
    return jnp.sum(pos_parts), jnp.sum(lse_parts), jnp.sum(picked_parts)


def _rare_ohem_branch(ops):
    pred, label, pos_num, neg_count, neg_sum = ops
    pred_value = jnp.max(pred[:, 1:], axis=1)
    is_neg = label == 0
    padded = jnp.where(is_neg, -pred_value, jnp.inf)
    sorted_neg_score = jnp.sort(padded)
    raw_idx = neg_sum - 1
    idx = jnp.where(raw_idx >= 0, raw_idx, neg_count + raw_idx)
    idx = jnp.clip(idx, 0, padded.shape[0] - 1)
    threshold = -sorted_neg_score[idx]
    mask = (pred_value >= threshold) | (label != 0)
    masked_label = jnp.where(mask, label, _IGNORE)
    logp = jax.nn.log_softmax(pred, axis=1)
    valid = masked_label != _IGNORE
    safe = jnp.where(valid, masked_label, 0)
    nll = -jnp.take_along_axis(logp, safe[:, None], axis=1)[:, 0]
    denom = jnp.maximum(jnp.sum(valid), 1).astype(pred.dtype)
    return jnp.sum(jnp.where(valid, nll, 0.0)) / denom


def kernel(pred, label):
    pos_num, sum_lse, sum_picked = _ce_pass(pred, label)
    neg_count = _N - pos_num
    neg_sum = pos_num * _FACTOR
    common = (sum_lse - sum_picked) / jnp.float32(_N)
    return lax.cond(
        neg_count > neg_sum,
        _rare_ohem_branch,
        lambda ops: common,
        (pred, label, pos_num, neg_count, neg_sum),
    )


# speedup vs baseline: 9.3394x; 5.5958x over previous
import jax
import jax.numpy as jnp
from jax import lax
from jax.experimental import pallas as pl
from jax.experimental.pallas import tpu as pltpu

_FACTOR = 3
_IGNORE = -100
_N = 262144
_C = 21
_BLK = 16384
_G = _N // _BLK


def _ce_pass_body(pred_ref, label_ref, pos_ref, lse_ref, picked_ref):
    x = pred_ref[...]
    lab = label_ref[0, 0, :]
    m = jnp.max(x)
    e = jnp.exp(x - m)
    s = jnp.sum(e, axis=0)
    lse = jnp.log(s) + m
    cls = lax.broadcasted_iota(jnp.int32, x.shape, 0)
    picked = jnp.sum(jnp.where(cls == lab[None, :], x, 0.0), axis=0)
    i = pl.program_id(0)
    pos_ref[0, i] = jnp.sum((lab != 0).astype(jnp.int32))
    lse_ref[0, i] = jnp.sum(lse)
    picked_ref[0, i] = jnp.sum(picked)


def _ce_pass(pred, label):
    pred_t = pred.T
    label3 = label.reshape(_G, 1, _BLK)
    out = pl.pallas_call(
        _ce_pass_body,
        grid=(_G,),
        in_specs=[
            pl.BlockSpec((_C, _BLK), lambda i: (0, i)),
            pl.BlockSpec((1, 1, _BLK), lambda i: (i, 0, 0)),
        ],
        out_specs=[
            pl.BlockSpec((1, _G), lambda i: (0, 0), memory_space=pltpu.SMEM),
            pl.BlockSpec((1, _G), lambda i: (0, 0), memory_space=pltpu.SMEM),
            pl.BlockSpec((1, _G), lambda i: (0, 0), memory_space=pltpu.SMEM),
        ],
        out_shape=[
            jax.ShapeDtypeStruct((1, _G), jnp.int32),
            jax.ShapeDtypeStruct((1, _G), jnp.float32),
            jax.ShapeDtypeStruct((1, _G), jnp.float32),
        ],
    )(pred_t, label3)
    pos_parts, lse_parts, picked_parts = out
    return jnp.sum(pos_parts), jnp.sum(lse_parts), jnp.sum(picked_parts)


def _rare_ohem_branch(ops):
    pred, label, pos_num, neg_count, neg_sum = ops
    pred_value = jnp.max(pred[:, 1:], axis=1)
    is_neg = label == 0
    padded = jnp.where(is_neg, -pred_value, jnp.inf)
    sorted_neg_score = jnp.sort(padded)
    raw_idx = neg_sum - 1
    idx = jnp.where(raw_idx >= 0, raw_idx, neg_count + raw_idx)
    idx = jnp.clip(idx, 0, padded.shape[0] - 1)
    threshold = -sorted_neg_score[idx]
    mask = (pred_value >= threshold) | (label != 0)
    masked_label = jnp.where(mask, label, _IGNORE)
    logp = jax.nn.log_softmax(pred, axis=1)
    valid = masked_label != _IGNORE
    safe = jnp.where(valid, masked_label, 0)
    nll = -jnp.take_along_axis(logp, safe[:, None], axis=1)[:, 0]
    denom = jnp.maximum(jnp.sum(valid), 1).astype(pred.dtype)
    return jnp.sum(jnp.where(valid, nll, 0.0)) / denom


def kernel(pred, label):
    pos_num, sum_lse, sum_picked = _ce_pass(pred, label)
    neg_count = _N - pos_num
    neg_sum = pos_num * _FACTOR
    common = (sum_lse - sum_picked) / jnp.float32(_N)
    return lax.cond(
        neg_count > neg_sum,
        _rare_ohem_branch,
        lambda ops: common,
        (pred, label, pos_num, neg_count, neg_sum),
    )
